# baseline (speedup 1.0000x reference)
import functools

import jax
import jax.numpy as jnp
from jax import lax
from jax.experimental import pallas as pl
from jax.experimental.pallas import tpu as pltpu
from jax.experimental.pallas import tpu_sc as plsc

N = 50000
E = 400000
H = 3
C = 32
HC = H * C
SEM = 64
R = 3

NPAD = 50176
BLK = 1024
SLAB = 12512
SLABZ = 12544
DUMMY = SLAB
EK = 128
NCH = E // EK


def _prep_body(x_ref, w_ref, a_ref, xp0_ref, xp1_ref, xp2_ref,
               t0_ref, t1_ref, t2_ref, t3_ref, t4_ref, t5_ref):
    xb = x_ref[...]
    xp = jnp.dot(xb, w_ref[...], preferred_element_type=jnp.float32,
                 precision=lax.Precision.HIGHEST)
    xp0_ref[...] = xp[:, 0 * HC:1 * HC]
    xp1_ref[...] = xp[:, 1 * HC:2 * HC]
    xp2_ref[...] = xp[:, 2 * HC:3 * HC]
    ts = jnp.dot(xp, a_ref[...], preferred_element_type=jnp.float32,
                 precision=lax.Precision.HIGHEST)
    for i, ref in enumerate((t0_ref, t1_ref, t2_ref, t3_ref, t4_ref, t5_ref)):
        ref[...] = ts[:, i * 16:(i + 1) * 16]


def _prep(xpad, wcat, acat):
    grid = (NPAD // BLK,)
    shp = [jax.ShapeDtypeStruct((NPAD, HC), jnp.float32)] * 3 + \
          [jax.ShapeDtypeStruct((NPAD, 16), jnp.float32)] * 6
    return pl.pallas_call(
        _prep_body,
        grid=grid,
        in_specs=[
            pl.BlockSpec((BLK, 128), lambda i: (i, 0)),
            pl.BlockSpec((128, R * HC), lambda i: (0, 0)),
            pl.BlockSpec((R * HC, 96), lambda i: (0, 0)),
        ],
        out_specs=[pl.BlockSpec((BLK, HC), lambda i: (i, 0))] * 3 +
                  [pl.BlockSpec((BLK, 16), lambda i: (i, 0))] * 6,
        out_shape=shp,
    )(xpad, wcat, acat)


_MESH = plsc.VectorSubcoreMesh(core_axis_name="c", subcore_axis_name="s")


@functools.partial(
    pl.kernel,
    out_type=jax.ShapeDtypeStruct((E, 16), jnp.float32),
    mesh=_MESH,
    compiler_params=pltpu.CompilerParams(use_tc_tiling_on_sc=False, needs_layout_passes=False),
    scratch_types=[
        pltpu.VMEM((EK,), jnp.int32),
        pltpu.VMEM((EK,), jnp.int32),
        pltpu.VMEM((EK, 16), jnp.float32),
        pltpu.VMEM((EK, 16), jnp.float32),
        pltpu.SemaphoreType.DMA,
        pltpu.SemaphoreType.DMA,
    ],
)
def _pass1(src_hbm, dst_hbm, ts_hbm, td_hbm, w_hbm,
           srcb, dstb, sbuf, dbuf, sem1, sem2):
    c = lax.axis_index("c")
    s = lax.axis_index("s")
    t = s * 2 + c
    cnt = (NCH // 32) + jnp.where(t < (NCH % 32), 1, 0)

    def chunk(i, carry):
        base = (t + 32 * i) * EK
        pltpu.sync_copy(src_hbm.at[pl.ds(base, EK)], srcb)
        pltpu.sync_copy(dst_hbm.at[pl.ds(base, EK)], dstb)
        cp1 = pltpu.async_copy(ts_hbm.at[srcb], sbuf, sem1)
        cp2 = pltpu.async_copy(td_hbm.at[dstb], dbuf, sem2)
        cp1.wait()
        cp2.wait()

        def edge(e, carry2):
            a = sbuf[e, :] + dbuf[e, :]
            a = jnp.where(a >= 0.0, a, 0.2 * a)
            sbuf[e, :] = jnp.exp(a)
            return carry2

        lax.fori_loop(0, EK, edge, 0)
        pltpu.sync_copy(sbuf, w_hbm.at[pl.ds(base, EK)])
        return carry

    lax.fori_loop(0, cnt, chunk, 0)


@functools.partial(
    pl.kernel,
    out_type=[jax.ShapeDtypeStruct((NPAD, HC), jnp.float32),
              jax.ShapeDtypeStruct((NPAD, 16), jnp.float32)],
    mesh=_MESH,
    compiler_params=pltpu.CompilerParams(use_tc_tiling_on_sc=False, needs_layout_passes=False),
    scratch_types=[
        pltpu.VMEM((EK,), jnp.int32),
        pltpu.VMEM((EK,), jnp.int32),
        pltpu.VMEM((EK,), jnp.int32),
        pltpu.VMEM((EK,), jnp.int32),
        pltpu.VMEM((EK,), jnp.int32),
        pltpu.VMEM((EK,), jnp.float32),
        pltpu.VMEM((EK,), jnp.float32),
        pltpu.VMEM((EK,), jnp.float32),
        pltpu.VMEM((EK, 16), jnp.float32),
        pltpu.VMEM((EK, HC), jnp.float32),
        pltpu.VMEM((112, HC), jnp.float32),
        pltpu.VMEM((112, 16), jnp.float32),
        pltpu.VMEM_SHARED((SLABZ, HC), jnp.float32),
        pltpu.VMEM_SHARED((SLABZ, 16), jnp.float32),
        pltpu.SemaphoreType.DMA,
    ],
)
def _pass2(src_hbm, dst_hbm, w_hbm, xp_hbm, acc_hbm, den_hbm,
           srcb, dstb, dlb, srcg, dlg, w0c, w1c, w2c,
           wb, xb, zacc, zden, accs, dens, sem):
    c = lax.axis_index("c")
    s = lax.axis_index("s")
    cnt = (NCH // 16) + jnp.where(s < (NCH % 16), 1, 0)
    zv = jnp.zeros((16,), jnp.float32)
    iota16 = lax.broadcasted_iota(jnp.int32, (16,), 0)
    h0 = jnp.zeros((16,), jnp.int32)
    h1 = jnp.full((16,), 1, jnp.int32)
    h2 = jnp.full((16,), 2, jnp.int32)
    dummyv = jnp.full((16,), DUMMY, jnp.int32)
    for j in range(EK // 16):
        srcg[pl.ds(j * 16, 16)] = jnp.zeros((16,), jnp.int32)

    def zfill(i, carry):
        for j in range(HC // 16):
            zacc[i, pl.ds(j * 16, 16)] = zv
        zden[i, :] = zv
        return carry

    lax.fori_loop(0, 112, zfill, 0)

    for sl in range(2):
        slab_id = 2 * sl + c
        lo = slab_id * SLAB
        hi = lo + SLAB
        for q in range(7):
            row0 = s * 784 + q * 112
            pltpu.sync_copy(zacc, accs.at[pl.ds(row0, 112)])
            pltpu.sync_copy(zden, dens.at[pl.ds(row0, 112)])
        plsc.subcore_barrier()

        def chunk(i, carry):
            base = (s + 16 * i) * EK
            pltpu.sync_copy(src_hbm.at[pl.ds(base, EK)], srcb)
            pltpu.sync_copy(dst_hbm.at[pl.ds(base, EK)], dstb)
            pltpu.sync_copy(w_hbm.at[pl.ds(base, EK)], wb)
            cnts = []
            for j in range(EK // 16):
                dlg[pl.ds(j * 16, 16)] = dummyv
                dv = dstb[pl.ds(j * 16, 16)]
                sv = srcb[pl.ds(j * 16, 16)]
                ev = j * 16 + iota16
                m = (dv >= lo) & (dv < hi)
                dl = dv - lo
                dlb[pl.ds(j * 16, 16)] = jnp.where(m, dl, DUMMY)
                w0 = plsc.load_gather(wb, [ev, h0])
                w1 = plsc.load_gather(wb, [ev, h1])
                w2 = plsc.load_gather(wb, [ev, h2])
                plsc.store_compressed(srcg.at[pl.ds(j * 16, 16)], sv, mask=m)
                plsc.store_compressed(dlg.at[pl.ds(j * 16, 16)], dl, mask=m)
                plsc.store_compressed(w0c.at[pl.ds(j * 16, 16)], w0, mask=m)
                plsc.store_compressed(w1c.at[pl.ds(j * 16, 16)], w1, mask=m)
                plsc.store_compressed(w2c.at[pl.ds(j * 16, 16)], w2, mask=m)
                cnts.append(jnp.sum(m.astype(jnp.int32)))

            pltpu.async_copy(xp_hbm.at[srcg], xb, sem).wait()

            for j in range(EK // 16):
                def edge(e, carry2, j=j):
                    eg = j * 16 + e
                    es = jnp.full((16,), eg, jnp.int32)
                    wv0 = plsc.load_gather(w0c, [es])
                    wv1 = plsc.load_gather(w1c, [es])
                    wv2 = plsc.load_gather(w2c, [es])
                    for h, wv in ((0, wv0), (1, wv1), (2, wv2)):
                        for k in range(C // 16):
                            off = h * C + k * 16
                            xb[eg, pl.ds(off, 16)] = xb[eg, pl.ds(off, 16)] * wv
                    return carry2

                lax.fori_loop(0, cnts[j], edge, 0)
            pltpu.sync_copy(xb, accs.at[dlg], add=True)
            pltpu.sync_copy(wb, dens.at[dlb], add=True)
            return carry

        lax.fori_loop(0, cnt, chunk, 0)
        plsc.subcore_barrier()
        row0 = s * 782
        pltpu.sync_copy(accs.at[pl.ds(row0, 782)],
                        acc_hbm.at[pl.ds(lo + row0, 782)])
        pltpu.sync_copy(dens.at[pl.ds(row0, 782)],
                        den_hbm.at[pl.ds(lo + row0, 782)])
        plsc.subcore_barrier()


def _final_body(a0_ref, a1_ref, a2_ref, d0_ref, d1_ref, d2_ref,
                bias_ref, w_ref, b_ref, q_ref, mg_ref, o_ref):
    outs = []
    for a_ref, d_ref, rr in ((a0_ref, d0_ref, 0), (a1_ref, d1_ref, 1),
                             (a2_ref, d2_ref, 2)):
        den = d_ref[...]
        denb = jnp.concatenate(
            [jnp.broadcast_to(den[:, h:h + 1], (den.shape[0], C))
             for h in range(H)], axis=1)
        o = a_ref[...] / (denb + 1e-16) + bias_ref[0, rr * HC:(rr + 1) * HC]
        outs.append(jnp.maximum(o, 0.0))
    wmat = w_ref[...]
    bvec = b_ref[...]
    qvec = q_ref[...]
    betas = []
    for rr in range(R):
        wr = jnp.tanh(jnp.dot(outs[rr], wmat,
                              preferred_element_type=jnp.float32,
                              precision=lax.Precision.HIGHEST) + bvec)
        betas.append(jnp.sum(qvec * wr, axis=-1, keepdims=True))
    bstack = jnp.concatenate(betas, axis=1)
    bmax = jnp.max(bstack, axis=1, keepdims=True)
    be = jnp.exp(bstack - bmax)
    bsum = jnp.sum(be, axis=1, keepdims=True)
    z = jnp.zeros_like(outs[0])
    for rr in range(R):
        z = z + outs[rr] * (be[:, rr:rr + 1] / bsum)
    o_ref[...] = z + mg_ref[...]


def _final(accs, dens, biascat, Wm, bv, qv, mg):
    grid = (NPAD // BLK,)
    return pl.pallas_call(
        _final_body,
        grid=grid,
        in_specs=[pl.BlockSpec((BLK, HC), lambda i: (i, 0))] * 3 +
                 [pl.BlockSpec((BLK, 16), lambda i: (i, 0))] * 3 +
                 [
                     pl.BlockSpec((1, R * HC), lambda i: (0, 0)),
                     pl.BlockSpec((HC, SEM), lambda i: (0, 0)),
                     pl.BlockSpec((1, SEM), lambda i: (0, 0)),
                     pl.BlockSpec((1, SEM), lambda i: (0, 0)),
                     pl.BlockSpec((1, HC), lambda i: (0, 0)),
                 ],
        out_specs=pl.BlockSpec((BLK, HC), lambda i: (i, 0)),
        out_shape=jax.ShapeDtypeStruct((NPAD, HC), jnp.float32),
    )(*accs, *dens, biascat, Wm, bv, qv, mg)


def kernel(x, edge_index0, edge_index1, edge_index2, lin_w0, att_src0, att_dst0, bias0, lin_w1, att_src1, att_dst1, bias1, lin_w2, att_src2, att_dst2, bias2, W, b, q, metagraph_row, g_att_src, g_att_dst):
    f32 = jnp.float32
    wcat = jnp.concatenate([lin_w0, lin_w1, lin_w2], axis=1)
    acat = jnp.zeros((R * HC, 96), f32)
    for rr, (asrc, adst) in enumerate(((att_src0, att_dst0),
                                       (att_src1, att_dst1),
                                       (att_src2, att_dst2))):
        a_s = asrc.reshape(H, C)
        a_d = adst.reshape(H, C)
        for h in range(H):
            acat = acat.at[rr * HC + h * C:rr * HC + (h + 1) * C,
                           rr * 32 + h].set(a_s[h])
            acat = acat.at[rr * HC + h * C:rr * HC + (h + 1) * C,
                           rr * 32 + 16 + h].set(a_d[h])

    xpad = jnp.pad(x, ((0, NPAD - N), (0, 0)))
    xp0, xp1, xp2, ts0, td0, ts1, td1, ts2, td2 = _prep(xpad, wcat, acat)

    accs, dens = [], []
    for src, dst, ts, td, xp in (
            (edge_index0[0], edge_index0[1], ts0, td0, xp0),
            (edge_index1[0], edge_index1[1], ts1, td1, xp1),
            (edge_index2[0], edge_index2[1], ts2, td2, xp2)):
        w_tab = _pass1(src, dst, ts, td)
        acc, den = _pass2(src, dst, w_tab, xp)
        accs.append(acc)
        dens.append(den)

    gamma = (g_att_src + g_att_dst).reshape(-1)
    mg = (metagraph_row * gamma).reshape(1, HC)
    biascat = jnp.concatenate([bias0, bias1, bias2]).reshape(1, R * HC)
    z = _final(accs, dens, biascat, W.reshape(HC, SEM), b.reshape(1, SEM),
               q.reshape(1, SEM), mg)
    return z[:N]


# speedup vs baseline: 7.6017x; 7.6017x over previous
import functools

import jax
import jax.numpy as jnp
from jax import lax
from jax.experimental import pallas as pl
from jax.experimental.pallas import tpu as pltpu
from jax.experimental.pallas import tpu_sc as plsc

N = 50000
E = 400000
H = 3
C = 32
HC = H * C
SEM = 64
R = 3

NPAD = 50176
BLK = 1024
SLAB = 12512
SLABZ = 12544
DUMMY = SLAB
EK = 128
NCH = E // EK


def _prep_body(x_ref, w_ref, a_ref, xp0_ref, xp1_ref, xp2_ref,
               t0_ref, t1_ref, t2_ref, t3_ref, t4_ref, t5_ref):
    xb = x_ref[...]
    xp = jnp.dot(xb, w_ref[...], preferred_element_type=jnp.float32,
                 precision=lax.Precision.HIGHEST)
    xp0_ref[...] = xp[:, 0 * HC:1 * HC]
    xp1_ref[...] = xp[:, 1 * HC:2 * HC]
    xp2_ref[...] = xp[:, 2 * HC:3 * HC]
    ts = jnp.dot(xp, a_ref[...], preferred_element_type=jnp.float32,
                 precision=lax.Precision.HIGHEST)
    for i, ref in enumerate((t0_ref, t1_ref, t2_ref, t3_ref, t4_ref, t5_ref)):
        ref[...] = ts[:, i * 16:(i + 1) * 16]


def _prep(xpad, wcat, acat):
    grid = (NPAD // BLK,)
    shp = [jax.ShapeDtypeStruct((NPAD, HC), jnp.float32)] * 3 + \
          [jax.ShapeDtypeStruct((NPAD, 16), jnp.float32)] * 6
    return pl.pallas_call(
        _prep_body,
        grid=grid,
        in_specs=[
            pl.BlockSpec((BLK, 128), lambda i: (i, 0)),
            pl.BlockSpec((128, R * HC), lambda i: (0, 0)),
            pl.BlockSpec((R * HC, 96), lambda i: (0, 0)),
        ],
        out_specs=[pl.BlockSpec((BLK, HC), lambda i: (i, 0))] * 3 +
                  [pl.BlockSpec((BLK, 16), lambda i: (i, 0))] * 6,
        out_shape=shp,
    )(xpad, wcat, acat)


_MESH = plsc.VectorSubcoreMesh(core_axis_name="c", subcore_axis_name="s")


@functools.partial(
    pl.kernel,
    out_type=jax.ShapeDtypeStruct((E, 16), jnp.float32),
    mesh=_MESH,
    compiler_params=pltpu.CompilerParams(use_tc_tiling_on_sc=False, needs_layout_passes=False),
    scratch_types=[
        pltpu.VMEM((EK,), jnp.int32),
        pltpu.VMEM((EK,), jnp.int32),
        pltpu.VMEM((EK, 16), jnp.float32),
        pltpu.VMEM((EK, 16), jnp.float32),
        pltpu.SemaphoreType.DMA,
        pltpu.SemaphoreType.DMA,
    ],
)
def _pass1(src_hbm, dst_hbm, ts_hbm, td_hbm, w_hbm,
           srcb, dstb, sbuf, dbuf, sem1, sem2):
    c = lax.axis_index("c")
    s = lax.axis_index("s")
    t = s * 2 + c
    cnt = (NCH // 32) + jnp.where(t < (NCH % 32), 1, 0)

    def chunk(i, carry):
        base = (t + 32 * i) * EK
        pltpu.sync_copy(src_hbm.at[pl.ds(base, EK)], srcb)
        pltpu.sync_copy(dst_hbm.at[pl.ds(base, EK)], dstb)
        cp1 = pltpu.async_copy(ts_hbm.at[srcb], sbuf, sem1)
        cp2 = pltpu.async_copy(td_hbm.at[dstb], dbuf, sem2)
        cp1.wait()
        cp2.wait()

        def edge(e, carry2):
            a = sbuf[e, :] + dbuf[e, :]
            a = jnp.where(a >= 0.0, a, 0.2 * a)
            sbuf[e, :] = jnp.exp(a)
            return carry2

        lax.fori_loop(0, EK, edge, 0)
        pltpu.sync_copy(sbuf, w_hbm.at[pl.ds(base, EK)])
        return carry

    lax.fori_loop(0, cnt, chunk, 0)


@functools.partial(
    pl.kernel,
    out_type=[jax.ShapeDtypeStruct((NPAD, HC), jnp.float32),
              jax.ShapeDtypeStruct((NPAD, 16), jnp.float32)],
    mesh=_MESH,
    compiler_params=pltpu.CompilerParams(use_tc_tiling_on_sc=False, needs_layout_passes=False),
    scratch_types=[
        pltpu.VMEM((EK,), jnp.int32),
        pltpu.VMEM((EK,), jnp.int32),
        pltpu.VMEM((EK,), jnp.int32),
        pltpu.VMEM((EK, 16), jnp.float32),
        pltpu.VMEM((EK, HC), jnp.float32),
        pltpu.VMEM((112, HC), jnp.float32),
        pltpu.VMEM((112, 16), jnp.float32),
        pltpu.VMEM_SHARED((SLABZ, HC), jnp.float32),
        pltpu.VMEM_SHARED((SLABZ, 16), jnp.float32),
        pltpu.SemaphoreType.DMA,
    ],
)
def _pass2(src_hbm, dst_hbm, w_hbm, xp_hbm, acc_hbm, den_hbm,
           srcb, dstb, dlb, wb, xb, zacc, zden, accs, dens, sem):
    c = lax.axis_index("c")
    s = lax.axis_index("s")
    cnt = (NCH // 16) + jnp.where(s < (NCH % 16), 1, 0)
    zv = jnp.zeros((16,), jnp.float32)
    dummy_row = SLAB + s

    def zfill(i, carry):
        for j in range(HC // 16):
            zacc[i, pl.ds(j * 16, 16)] = zv
        zden[i, :] = zv
        return carry

    lax.fori_loop(0, 112, zfill, 0)

    for sl in range(2):
        slab_id = 2 * sl + c
        lo = slab_id * SLAB
        hi = lo + SLAB
        for q in range(7):
            row0 = s * 784 + q * 112
            pltpu.sync_copy(zacc, accs.at[pl.ds(row0, 112)])
            pltpu.sync_copy(zden, dens.at[pl.ds(row0, 112)])
        plsc.subcore_barrier()

        def chunk(i, carry):
            base = (s + 16 * i) * EK
            pltpu.sync_copy(src_hbm.at[pl.ds(base, EK)], srcb)
            pltpu.sync_copy(dst_hbm.at[pl.ds(base, EK)], dstb)
            pltpu.sync_copy(w_hbm.at[pl.ds(base, EK)], wb)

            def vmask(j, carry2):
                dv = dstb[pl.ds(j * 16, 16)]
                m = (dv >= lo) & (dv < hi)
                dlb[pl.ds(j * 16, 16)] = jnp.where(m, dv - lo, dummy_row)
                return carry2

            lax.fori_loop(0, EK // 16, vmask, 0)
            pltpu.async_copy(xp_hbm.at[srcb], xb, sem).wait()

            def edge(e, carry2):
                es = jnp.full((16,), e, jnp.int32)
                for h in range(H):
                    wv = plsc.load_gather(
                        wb, [es, jnp.full((16,), h, jnp.int32)])
                    for k in range(C // 16):
                        off = h * C + k * 16
                        xb[e, pl.ds(off, 16)] = xb[e, pl.ds(off, 16)] * wv
                return carry2

            lax.fori_loop(0, EK, edge, 0)
            pltpu.sync_copy(xb, accs.at[dlb], add=True)
            pltpu.sync_copy(wb, dens.at[dlb], add=True)
            return carry

        lax.fori_loop(0, cnt, chunk, 0)
        plsc.subcore_barrier()
        row0 = s * 782
        pltpu.sync_copy(accs.at[pl.ds(row0, 782)],
                        acc_hbm.at[pl.ds(lo + row0, 782)])
        pltpu.sync_copy(dens.at[pl.ds(row0, 782)],
                        den_hbm.at[pl.ds(lo + row0, 782)])
        plsc.subcore_barrier()


def _final_body(a0_ref, a1_ref, a2_ref, d0_ref, d1_ref, d2_ref,
                bias_ref, w_ref, b_ref, q_ref, mg_ref, o_ref):
    outs = []
    for a_ref, d_ref, rr in ((a0_ref, d0_ref, 0), (a1_ref, d1_ref, 1),
                             (a2_ref, d2_ref, 2)):
        den = d_ref[...]
        denb = jnp.concatenate(
            [jnp.broadcast_to(den[:, h:h + 1], (den.shape[0], C))
             for h in range(H)], axis=1)
        o = a_ref[...] / (denb + 1e-16) + bias_ref[0, rr * HC:(rr + 1) * HC]
        outs.append(jnp.maximum(o, 0.0))
    wmat = w_ref[...]
    bvec = b_ref[...]
    qvec = q_ref[...]
    betas = []
    for rr in range(R):
        wr = jnp.tanh(jnp.dot(outs[rr], wmat,
                              preferred_element_type=jnp.float32,
                              precision=lax.Precision.HIGHEST) + bvec)
        betas.append(jnp.sum(qvec * wr, axis=-1, keepdims=True))
    bstack = jnp.concatenate(betas, axis=1)
    bmax = jnp.max(bstack, axis=1, keepdims=True)
    be = jnp.exp(bstack - bmax)
    bsum = jnp.sum(be, axis=1, keepdims=True)
    z = jnp.zeros_like(outs[0])
    for rr in range(R):
        z = z + outs[rr] * (be[:, rr:rr + 1] / bsum)
    o_ref[...] = z + mg_ref[...]


def _final(accs, dens, biascat, Wm, bv, qv, mg):
    grid = (NPAD // BLK,)
    return pl.pallas_call(
        _final_body,
        grid=grid,
        in_specs=[pl.BlockSpec((BLK, HC), lambda i: (i, 0))] * 3 +
                 [pl.BlockSpec((BLK, 16), lambda i: (i, 0))] * 3 +
                 [
                     pl.BlockSpec((1, R * HC), lambda i: (0, 0)),
                     pl.BlockSpec((HC, SEM), lambda i: (0, 0)),
                     pl.BlockSpec((1, SEM), lambda i: (0, 0)),
                     pl.BlockSpec((1, SEM), lambda i: (0, 0)),
                     pl.BlockSpec((1, HC), lambda i: (0, 0)),
                 ],
        out_specs=pl.BlockSpec((BLK, HC), lambda i: (i, 0)),
        out_shape=jax.ShapeDtypeStruct((NPAD, HC), jnp.float32),
    )(*accs, *dens, biascat, Wm, bv, qv, mg)


def kernel(x, edge_index0, edge_index1, edge_index2, lin_w0, att_src0, att_dst0, bias0, lin_w1, att_src1, att_dst1, bias1, lin_w2, att_src2, att_dst2, bias2, W, b, q, metagraph_row, g_att_src, g_att_dst):
    f32 = jnp.float32
    wcat = jnp.concatenate([lin_w0, lin_w1, lin_w2], axis=1)
    acat = jnp.zeros((R * HC, 96), f32)
    for rr, (asrc, adst) in enumerate(((att_src0, att_dst0),
                                       (att_src1, att_dst1),
                                       (att_src2, att_dst2))):
        a_s = asrc.reshape(H, C)
        a_d = adst.reshape(H, C)
        for h in range(H):
            acat = acat.at[rr * HC + h * C:rr * HC + (h + 1) * C,
                           rr * 32 + h].set(a_s[h])
            acat = acat.at[rr * HC + h * C:rr * HC + (h + 1) * C,
                           rr * 32 + 16 + h].set(a_d[h])

    xpad = jnp.pad(x, ((0, NPAD - N), (0, 0)))
    xp0, xp1, xp2, ts0, td0, ts1, td1, ts2, td2 = _prep(xpad, wcat, acat)

    accs, dens = [], []
    for src, dst, ts, td, xp in (
            (edge_index0[0], edge_index0[1], ts0, td0, xp0),
            (edge_index1[0], edge_index1[1], ts1, td1, xp1),
            (edge_index2[0], edge_index2[1], ts2, td2, xp2)):
        w_tab = _pass1(src, dst, ts, td)
        acc, den = _pass2(src, dst, w_tab, xp)
        accs.append(acc)
        dens.append(den)

    gamma = (g_att_src + g_att_dst).reshape(-1)
    mg = (metagraph_row * gamma).reshape(1, HC)
    biascat = jnp.concatenate([bias0, bias1, bias2]).reshape(1, R * HC)
    z = _final(accs, dens, biascat, W.reshape(HC, SEM), b.reshape(1, SEM),
               q.reshape(1, SEM), mg)
    return z[:N]


# speedup vs baseline: 8.3138x; 1.0937x over previous
import functools

import jax
import jax.numpy as jnp
from jax import lax
from jax.experimental import pallas as pl
from jax.experimental.pallas import tpu as pltpu
from jax.experimental.pallas import tpu_sc as plsc

N = 50000
E = 400000
H = 3
C = 32
HC = H * C
SEM = 64
R = 3

NPAD = 50176
BLK = 1024
SLAB = 12512
SLABZ = 12544
DUMMY = SLAB
EK = 80
EKL = 320
EPAD = 404480
EK1 = 128
NCH1 = EPAD // EK1
NGRP = EPAD // EKL
GPT = NGRP // 16


def _prep_body(x_ref, w_ref, a_ref, xp0_ref, xp1_ref, xp2_ref,
               t0_ref, t1_ref, t2_ref, t3_ref, t4_ref, t5_ref):
    xb = x_ref[...]
    xp = jnp.dot(xb, w_ref[...], preferred_element_type=jnp.float32,
                 precision=lax.Precision.HIGHEST)
    xp0_ref[...] = xp[:, 0 * HC:1 * HC]
    xp1_ref[...] = xp[:, 1 * HC:2 * HC]
    xp2_ref[...] = xp[:, 2 * HC:3 * HC]
    ts = jnp.dot(xp, a_ref[...], preferred_element_type=jnp.float32,
                 precision=lax.Precision.HIGHEST)
    for i, ref in enumerate((t0_ref, t1_ref, t2_ref, t3_ref, t4_ref, t5_ref)):
        ref[...] = ts[:, i * 16:(i + 1) * 16]


def _prep(xpad, wcat, acat):
    grid = (NPAD // BLK,)
    shp = [jax.ShapeDtypeStruct((NPAD, HC), jnp.float32)] * 3 + \
          [jax.ShapeDtypeStruct((NPAD, 16), jnp.float32)] * 6
    return pl.pallas_call(
        _prep_body,
        grid=grid,
        in_specs=[
            pl.BlockSpec((BLK, 128), lambda i: (i, 0)),
            pl.BlockSpec((128, R * HC), lambda i: (0, 0)),
            pl.BlockSpec((R * HC, 96), lambda i: (0, 0)),
        ],
        out_specs=[pl.BlockSpec((BLK, HC), lambda i: (i, 0))] * 3 +
                  [pl.BlockSpec((BLK, 16), lambda i: (i, 0))] * 6,
        out_shape=shp,
    )(xpad, wcat, acat)


_MESH = plsc.VectorSubcoreMesh(core_axis_name="c", subcore_axis_name="s")


@functools.partial(
    pl.kernel,
    out_type=jax.ShapeDtypeStruct((EPAD, 16), jnp.float32),
    mesh=_MESH,
    compiler_params=pltpu.CompilerParams(use_tc_tiling_on_sc=False, needs_layout_passes=False),
    scratch_types=[
        pltpu.VMEM((EK1,), jnp.int32),
        pltpu.VMEM((EK1,), jnp.int32),
        pltpu.VMEM((EK1, 16), jnp.float32),
        pltpu.VMEM((EK1, 16), jnp.float32),
        pltpu.SemaphoreType.DMA,
        pltpu.SemaphoreType.DMA,
    ],
)
def _pass1(src_hbm, dst_hbm, ts_hbm, td_hbm, w_hbm,
           srcb, dstb, sbuf, dbuf, sem1, sem2):
    c = lax.axis_index("c")
    s = lax.axis_index("s")
    t = s * 2 + c
    cnt = (NCH1 // 32) + jnp.where(t < (NCH1 % 32), 1, 0)

    def chunk(i, carry):
        base = (t + 32 * i) * EK1
        pltpu.sync_copy(src_hbm.at[pl.ds(base, EK1)], srcb)
        pltpu.sync_copy(dst_hbm.at[pl.ds(base, EK1)], dstb)
        cp1 = pltpu.async_copy(ts_hbm.at[srcb], sbuf, sem1)
        cp2 = pltpu.async_copy(td_hbm.at[dstb], dbuf, sem2)
        cp1.wait()
        cp2.wait()

        def edge(e, carry2):
            a = sbuf[e, :] + dbuf[e, :]
            a = jnp.where(a >= 0.0, a, 0.2 * a)
            sbuf[e, :] = jnp.exp(a)
            return carry2

        lax.fori_loop(0, EK1, edge, 0)
        pltpu.sync_copy(sbuf, w_hbm.at[pl.ds(base, EK1)])
        return carry

    lax.fori_loop(0, cnt, chunk, 0)


@functools.partial(
    pl.kernel,
    out_type=[jax.ShapeDtypeStruct((NPAD, HC), jnp.float32),
              jax.ShapeDtypeStruct((NPAD, 16), jnp.float32)],
    mesh=_MESH,
    compiler_params=pltpu.CompilerParams(use_tc_tiling_on_sc=False, needs_layout_passes=False),
    scratch_types=[
        pltpu.VMEM((EKL,), jnp.int32),
        pltpu.VMEM((EKL,), jnp.int32),
        pltpu.VMEM((EKL, 16), jnp.float32),
        pltpu.VMEM((EK,), jnp.int32),
        pltpu.VMEM((EK,), jnp.int32),
        pltpu.VMEM((EK,), jnp.int32),
        pltpu.VMEM((EK,), jnp.int32),
        pltpu.VMEM((EK,), jnp.int32),
        pltpu.VMEM((EK,), jnp.int32),
        pltpu.VMEM((EK,), jnp.int32),
        pltpu.VMEM((EK,), jnp.int32),
        pltpu.VMEM((EK, HC), jnp.float32),
        pltpu.VMEM((EK, HC), jnp.float32),
        pltpu.VMEM((EK, HC), jnp.float32),
        pltpu.VMEM((16, HC), jnp.float32),
        pltpu.VMEM((16, 16), jnp.float32),
        pltpu.VMEM_SHARED((SLABZ, HC), jnp.float32),
        pltpu.VMEM_SHARED((SLABZ, 16), jnp.float32),
        pltpu.SemaphoreType.DMA,
        pltpu.SemaphoreType.DMA,
        pltpu.SemaphoreType.DMA,
        pltpu.SemaphoreType.DMA,
        pltpu.SemaphoreType.DMA,
        pltpu.SemaphoreType.DMA,
        pltpu.SemaphoreType.DMA,
        pltpu.SemaphoreType.DMA,
        pltpu.SemaphoreType.DMA,
    ],
)
def _pass2(src_hbm, dst_hbm, w_hbm, xp_hbm, acc_hbm, den_hbm,
           srcb4, dstb4, wb4, si0, si1, si2, si3, dl0, dl1, dl2, dl3,
           xb0, xb1, xb2, zacc, zden, accs, dens,
           gs0, gs1, gs2, ss0, ss1, ss2, ss3, dsem, zsem):
    srci = (si0, si1, si2, si3)
    dlb = (dl0, dl1, dl2, dl3)
    xb = (xb0, xb1, xb2, xb0)
    gsem = (gs0, gs1, gs2, gs0)
    ssem = (ss0, ss1, ss2, ss3)
    c = lax.axis_index("c")
    s = lax.axis_index("s")
    zv = jnp.zeros((16,), jnp.float32)
    iota16 = lax.broadcasted_iota(jnp.int32, (16,), 0)
    dummy_row = SLAB + s

    def zfill(i, carry):
        for j in range(HC // 16):
            zacc[i, pl.ds(j * 16, 16)] = zv
        zden[i, :] = zv
        return carry

    lax.fori_loop(0, 16, zfill, 0)

    for sl in range(2):
        slab_id = 2 * sl + c
        lo = slab_id * SLAB
        hi = lo + SLAB
        row0s = s * 784

        def zcp(q, carry):
            pltpu.async_copy(zacc, accs.at[pl.ds(row0s + q * 16, 16)], zsem)
            pltpu.async_copy(zden, dens.at[pl.ds(row0s + q * 16, 16)], zsem)
            return carry

        lax.fori_loop(0, 49, zcp, 0)

        def zdr(q, carry):
            pltpu.make_async_copy(zacc, accs.at[pl.ds(row0s, 16)], zsem).wait()
            pltpu.make_async_copy(zden, dens.at[pl.ds(row0s, 16)], zsem).wait()
            return carry

        lax.fori_loop(0, 49, zdr, 0)
        plsc.subcore_barrier()

        def group(i, carry):
            base = (s + 16 * i) * EKL
            pltpu.sync_copy(src_hbm.at[pl.ds(base, EKL)], srcb4)
            pltpu.sync_copy(dst_hbm.at[pl.ds(base, EKL)], dstb4)
            pltpu.sync_copy(w_hbm.at[pl.ds(base, EKL)], wb4)

            for k in range(4):
                def prep(j, carry2, k=k):
                    o = k * EK + j * 16
                    sv = srcb4[pl.ds(o, 16)]
                    dv = dstb4[pl.ds(o, 16)]
                    srci[k][pl.ds(j * 16, 16)] = sv
                    ev = (base + o) + iota16
                    m = (dv >= lo) & (dv < hi) & (ev < E)
                    dlb[k][pl.ds(j * 16, 16)] = jnp.where(m, dv - lo,
                                                          dummy_row)
                    return carry2

                lax.fori_loop(0, EK // 16, prep, 0)

            gth = [pltpu.async_copy(xp_hbm.at[srci[k]], xb[k], gsem[k])
                   for k in range(3)]
            gth.append(None)
            sc_pend = []
            den_pend = []
            for k in range(4):
                gth[k].wait()

                def edge(e, carry2, k=k):
                    es = jnp.full((16,), k * EK + e, jnp.int32)
                    for h in range(H):
                        wv = plsc.load_gather(
                            wb4, [es, jnp.full((16,), h, jnp.int32)])
                        for q in range(C // 16):
                            off = h * C + q * 16
                            xb[k][e, pl.ds(off, 16)] = (
                                xb[k][e, pl.ds(off, 16)] * wv)
                    return carry2

                lax.fori_loop(0, EK, edge, 0)
                sc_pend.append(pltpu.async_copy(
                    xb[k], accs.at[dlb[k]], ssem[k], add=True))
                den_pend.append(pltpu.async_copy(
                    wb4.at[pl.ds(k * EK, EK)], dens.at[dlb[k]], dsem,
                    add=True))
                if k == 1:
                    sc_pend[0].wait()
                    gth[3] = pltpu.async_copy(xp_hbm.at[srci[3]], xb[3],
                                              gsem[3])
            for p in sc_pend[1:] + den_pend:
                p.wait()
            return carry

        lax.fori_loop(0, GPT, group, 0)
        plsc.subcore_barrier()
        row0 = s * 782
        pltpu.sync_copy(accs.at[pl.ds(row0, 782)],
                        acc_hbm.at[pl.ds(lo + row0, 782)])
        pltpu.sync_copy(dens.at[pl.ds(row0, 782)],
                        den_hbm.at[pl.ds(lo + row0, 782)])
        plsc.subcore_barrier()


def _final_body(a0_ref, a1_ref, a2_ref, d0_ref, d1_ref, d2_ref,
                bias_ref, w_ref, b_ref, q_ref, mg_ref, o_ref):
    outs = []
    for a_ref, d_ref, rr in ((a0_ref, d0_ref, 0), (a1_ref, d1_ref, 1),
                             (a2_ref, d2_ref, 2)):
        den = d_ref[...]
        denb = jnp.concatenate(
            [jnp.broadcast_to(den[:, h:h + 1], (den.shape[0], C))
             for h in range(H)], axis=1)
        o = a_ref[...] / (denb + 1e-16) + bias_ref[0, rr * HC:(rr + 1) * HC]
        outs.append(jnp.maximum(o, 0.0))
    wmat = w_ref[...]
    bvec = b_ref[...]
    qvec = q_ref[...]
    betas = []
    for rr in range(R):
        wr = jnp.tanh(jnp.dot(outs[rr], wmat,
                              preferred_element_type=jnp.float32,
                              precision=lax.Precision.HIGHEST) + bvec)
        betas.append(jnp.sum(qvec * wr, axis=-1, keepdims=True))
    bstack = jnp.concatenate(betas, axis=1)
    bmax = jnp.max(bstack, axis=1, keepdims=True)
    be = jnp.exp(bstack - bmax)
    bsum = jnp.sum(be, axis=1, keepdims=True)
    z = jnp.zeros_like(outs[0])
    for rr in range(R):
        z = z + outs[rr] * (be[:, rr:rr + 1] / bsum)
    o_ref[...] = z + mg_ref[...]


def _final(accs, dens, biascat, Wm, bv, qv, mg):
    grid = (NPAD // BLK,)
    return pl.pallas_call(
        _final_body,
        grid=grid,
        in_specs=[pl.BlockSpec((BLK, HC), lambda i: (i, 0))] * 3 +
                 [pl.BlockSpec((BLK, 16), lambda i: (i, 0))] * 3 +
                 [
                     pl.BlockSpec((1, R * HC), lambda i: (0, 0)),
                     pl.BlockSpec((HC, SEM), lambda i: (0, 0)),
                     pl.BlockSpec((1, SEM), lambda i: (0, 0)),
                     pl.BlockSpec((1, SEM), lambda i: (0, 0)),
                     pl.BlockSpec((1, HC), lambda i: (0, 0)),
                 ],
        out_specs=pl.BlockSpec((BLK, HC), lambda i: (i, 0)),
        out_shape=jax.ShapeDtypeStruct((NPAD, HC), jnp.float32),
    )(*accs, *dens, biascat, Wm, bv, qv, mg)


def kernel(x, edge_index0, edge_index1, edge_index2, lin_w0, att_src0, att_dst0, bias0, lin_w1, att_src1, att_dst1, bias1, lin_w2, att_src2, att_dst2, bias2, W, b, q, metagraph_row, g_att_src, g_att_dst):
    f32 = jnp.float32
    wcat = jnp.concatenate([lin_w0, lin_w1, lin_w2], axis=1)
    acat = jnp.zeros((R * HC, 96), f32)
    for rr, (asrc, adst) in enumerate(((att_src0, att_dst0),
                                       (att_src1, att_dst1),
                                       (att_src2, att_dst2))):
        a_s = asrc.reshape(H, C)
        a_d = adst.reshape(H, C)
        for h in range(H):
            acat = acat.at[rr * HC + h * C:rr * HC + (h + 1) * C,
                           rr * 32 + h].set(a_s[h])
            acat = acat.at[rr * HC + h * C:rr * HC + (h + 1) * C,
                           rr * 32 + 16 + h].set(a_d[h])

    xpad = jnp.pad(x, ((0, NPAD - N), (0, 0)))
    xp0, xp1, xp2, ts0, td0, ts1, td1, ts2, td2 = _prep(xpad, wcat, acat)

    accs, dens = [], []
    for ei, ts, td, xp in ((edge_index0, ts0, td0, xp0),
                           (edge_index1, ts1, td1, xp1),
                           (edge_index2, ts2, td2, xp2)):
        eip = jnp.pad(ei, ((0, 0), (0, EPAD - E)))
        srcv, dstv = eip[0], eip[1]
        w_tab = _pass1(srcv, dstv, ts, td)
        acc, den = _pass2(srcv, dstv, w_tab, xp)
        accs.append(acc)
        dens.append(den)

    gamma = (g_att_src + g_att_dst).reshape(-1)
    mg = (metagraph_row * gamma).reshape(1, HC)
    biascat = jnp.concatenate([bias0, bias1, bias2]).reshape(1, R * HC)
    z = _final(accs, dens, biascat, W.reshape(HC, SEM), b.reshape(1, SEM),
               q.reshape(1, SEM), mg)
    return z[:N]
